# baseline (speedup 1.0000x reference)
import functools

import jax
import jax.numpy as jnp
import numpy as np
from jax import lax
from jax.experimental import pallas as pl
from jax.experimental.pallas import tpu as pltpu
from jax.experimental.pallas import tpu_sc as plsc

B = 16384
F = 26
D = 64
N = F + 1
VOCAB = 100000

NC = 2
NS = 16
NW = NC * NS
ROWS = B * F
RPW = ROWS // NW
CH = 128
NCH = RPW // CH
NBUF = 4
NOUT = NCH // NBUF

_TI, _TJ = np.tril_indices(N, -1)


def _sc_gather(table2, idx3):
    mesh = plsc.VectorSubcoreMesh(core_axis_name="c", subcore_axis_name="s")

    @functools.partial(
        pl.kernel,
        out_type=jax.ShapeDtypeStruct((ROWS, 128), jnp.float32),
        mesh=mesh,
        scratch_types=[
            pltpu.VMEM((NCH, CH), jnp.int32),
            pltpu.VMEM((NBUF, CH, 128), jnp.float32),
        ]
        + [pltpu.SemaphoreType.DMA] * (2 * NBUF),
        compiler_params=pltpu.CompilerParams(use_tc_tiling_on_sc=True),
    )
    def k(table_hbm, idx_hbm, out_hbm, idx_v, rows_v, *sems):
        gs, ws = sems[:NBUF], sems[NBUF:]
        wid = lax.axis_index("s") * NC + lax.axis_index("c")
        base = wid * RPW
        pltpu.sync_copy(idx_hbm.at[wid], idx_v)

        def start_gather(j, s):
            pltpu.async_copy(table_hbm.at[idx_v.at[j]], rows_v.at[s], gs[s])

        def wait_gather(s):
            pltpu.make_async_copy(
                table_hbm.at[pl.ds(0, CH)], rows_v.at[s], gs[s]
            ).wait()

        def start_wb(j, s):
            pltpu.async_copy(
                rows_v.at[s], out_hbm.at[pl.ds(base + j * CH, CH)], ws[s]
            )

        def wait_wb(s):
            pltpu.make_async_copy(
                rows_v.at[s], out_hbm.at[pl.ds(0, CH)], ws[s]
            ).wait()

        for s in range(NBUF):
            start_gather(s, s)

        def outer(jj, carry):
            j0 = jj * NBUF
            for s in range(NBUF):
                wait_gather(s)
                start_wb(j0 + s, s)
            for s in range(NBUF):
                wait_wb(s)
                start_gather(j0 + NBUF + s, s)
            return carry

        lax.fori_loop(0, NOUT - 1, outer, 0)

        j0 = (NOUT - 1) * NBUF
        for s in range(NBUF):
            wait_gather(s)
            start_wb(j0 + s, s)
        for s in range(NBUF):
            wait_wb(s)

    return k(table2, idx3)


def _tc_body(df_ref, emb_ref, par_ref, wd1, bd1, wd2, bd2, wd3, bd3,
             w1d, w1e, bo1, wo2, bo2, wo3, bo3, wo4, bo4, wo5, bo5,
             out_ref, *, bb):
    f32 = jnp.float32
    x = df_ref[...]
    d = jax.nn.relu(jnp.dot(x, wd1[...], preferred_element_type=f32) + bd1[...])
    d = jax.nn.relu(jnp.dot(d, wd2[...], preferred_element_type=f32) + bd2[...])
    d = jax.nn.relu(jnp.dot(d, wd3[...], preferred_element_type=f32) + bd3[...])

    pair = emb_ref[...]
    p = par_ref[...]
    low = pair[:, :D]
    high = pair[:, D:]
    em = low + p * (high - low)
    emb3 = em.reshape(bb, F, D)
    comb = jnp.concatenate([d[:, None, :], emb3], axis=1)
    g = lax.dot_general(
        comb, comb, (((2,), (2,)), ((0,), (0,))),
        preferred_element_type=f32,
    )
    gf = g.reshape(bb, N * N)

    h = jnp.dot(d, w1d[...], preferred_element_type=f32)
    h = h + jnp.dot(gf, w1e[...], preferred_element_type=f32)
    h = jax.nn.relu(h + bo1[...])
    h = jax.nn.relu(jnp.dot(h, wo2[...], preferred_element_type=f32) + bo2[...])
    h = jax.nn.relu(jnp.dot(h, wo3[...], preferred_element_type=f32) + bo3[...])
    h = jax.nn.relu(jnp.dot(h, wo4[...], preferred_element_type=f32) + bo4[...])
    out_ref[...] = jnp.sum(h * wo5[...], axis=1, keepdims=True) + bo5[...]


def kernel(dense_features, sparse_indices, offsets, W_embed, dense_params, over_params):
    flat_idx = (sparse_indices + offsets[None, :]).astype(jnp.int32).reshape(-1)
    idx3 = (flat_idx >> 1).reshape(NW, NCH, CH)
    par = (flat_idx & 1).astype(jnp.float32).reshape(ROWS, 1)
    table2 = W_embed.reshape(VOCAB * F // 2, 128)
    pairs = _sc_gather(table2, idx3)

    dfp = jnp.pad(dense_features, ((0, 0), (0, 3)))
    (wd1, bd1), (wd2, bd2), (wd3, bd3) = dense_params
    wd1p = jnp.pad(wd1, ((0, 3), (0, 0)))
    (wo1, bo1), (wo2, bo2), (wo3, bo3), (wo4, bo4), (wo5, bo5) = over_params

    w1d = wo1[:D]
    w1e = jnp.zeros((N * N, wo1.shape[1]), jnp.float32)
    w1e = w1e.at[_TI * N + _TJ].set(wo1[D:])

    bb = 512
    nblk = B // bb
    row2 = lambda a: a.reshape(1, -1)

    grid_spec = pl.GridSpec(
        grid=(nblk,),
        in_specs=[
            pl.BlockSpec((bb, 16), lambda i: (i, 0)),
            pl.BlockSpec((bb * F, 128), lambda i: (i, 0)),
            pl.BlockSpec((bb * F, 1), lambda i: (i, 0)),
        ]
        + [pl.BlockSpec(s, lambda i: (0, 0)) for s in [
            (16, 512), (1, 512), (512, 256), (1, 256), (256, 64), (1, 64),
            (64, 1024), (N * N, 1024), (1, 1024), (1024, 1024), (1, 1024),
            (1024, 512), (1, 512), (512, 256), (1, 256), (1, 256), (1, 1),
        ]],
        out_specs=pl.BlockSpec((bb, 1), lambda i: (i, 0)),
    )
    logits = pl.pallas_call(
        functools.partial(_tc_body, bb=bb),
        grid_spec=grid_spec,
        out_shape=jax.ShapeDtypeStruct((B, 1), jnp.float32),
        compiler_params=pltpu.CompilerParams(
            dimension_semantics=("arbitrary",),
        ),
    )(
        dfp, pairs, par,
        wd1p, row2(bd1), wd2, row2(bd2), wd3, row2(bd3),
        w1d, w1e, row2(bo1), wo2, row2(bo2), wo3, row2(bo3),
        wo4, row2(bo4), row2(wo5[:, 0]), bo5.reshape(1, 1),
    )
    return logits


# speedup vs baseline: 1.1265x; 1.1265x over previous
import functools

import jax
import jax.numpy as jnp
import numpy as np
from jax import lax
from jax.experimental import pallas as pl
from jax.experimental.pallas import tpu as pltpu
from jax.experimental.pallas import tpu_sc as plsc

B = 16384
F = 26
D = 64
N = F + 1
VOCAB = 100000

NC = 2
NS = 16
NW = NC * NS
ROWS = B * F
RPW = ROWS // NW
CH = 128
NCH = RPW // CH
NBUF = 4
NOUT = NCH // NBUF

_POS = np.zeros(N, np.int64)
_POS[0] = 0
_POS[1::2] = 1 + np.arange(F // 2)
_POS[2::2] = 1 + F // 2 + np.arange(F // 2)
_TI, _TJ = np.tril_indices(N, -1)
_SCAT = _POS[_TI] * N + _POS[_TJ]


def _sc_gather(table, idx3):
    mesh = plsc.VectorSubcoreMesh(core_axis_name="c", subcore_axis_name="s")

    @functools.partial(
        pl.kernel,
        out_type=jax.ShapeDtypeStruct((ROWS, D), jnp.float32),
        mesh=mesh,
        scratch_types=[
            pltpu.VMEM((NCH, CH), jnp.int32),
            pltpu.VMEM((NBUF, CH, D), jnp.float32),
        ]
        + [pltpu.SemaphoreType.DMA] * (2 * NBUF),
        compiler_params=pltpu.CompilerParams(use_tc_tiling_on_sc=False),
    )
    def k(table_hbm, idx_hbm, out_hbm, idx_v, rows_v, *sems):
        gs, ws = sems[:NBUF], sems[NBUF:]
        wid = lax.axis_index("s") * NC + lax.axis_index("c")
        base = wid * RPW
        pltpu.sync_copy(idx_hbm.at[wid], idx_v)

        def start_gather(j, s):
            pltpu.async_copy(table_hbm.at[idx_v.at[j]], rows_v.at[s], gs[s])

        def wait_gather(s):
            pltpu.make_async_copy(
                table_hbm.at[pl.ds(0, CH)], rows_v.at[s], gs[s]
            ).wait()

        def start_wb(j, s):
            pltpu.async_copy(
                rows_v.at[s], out_hbm.at[pl.ds(base + j * CH, CH)], ws[s]
            )

        def wait_wb(s):
            pltpu.make_async_copy(
                rows_v.at[s], out_hbm.at[pl.ds(0, CH)], ws[s]
            ).wait()

        for s in range(NBUF):
            start_gather(s, s)

        def outer(jj, carry):
            j0 = jj * NBUF
            for s in range(NBUF):
                wait_gather(s)
                start_wb(j0 + s, s)
            for s in range(NBUF):
                wait_wb(s)
                start_gather(j0 + NBUF + s, s)
            return carry

        lax.fori_loop(0, NOUT - 1, outer, 0)

        j0 = (NOUT - 1) * NBUF
        for s in range(NBUF):
            wait_gather(s)
            start_wb(j0 + s, s)
        for s in range(NBUF):
            wait_wb(s)

    return k(table, idx3)


def _tc_body(df_ref, emb_ref, wd1, bd1, wd2, bd2, wd3, bd3,
             w1d, w1e, bo1, wo2, bo2, wo3, bo3, wo4, bo4, wo5, bo5,
             out_ref, *, bb):
    f32 = jnp.float32
    bf16 = jnp.bfloat16
    x = df_ref[...]
    d = jax.nn.relu(jnp.dot(x, wd1[...], preferred_element_type=f32) + bd1[...])
    d = jax.nn.relu(
        jnp.dot(d.astype(bf16), wd2[...], preferred_element_type=f32) + bd2[...])
    d = jax.nn.relu(
        jnp.dot(d.astype(bf16), wd3[...], preferred_element_type=f32) + bd3[...])
    db = d.astype(bf16)

    pm = emb_ref[...].astype(bf16)
    em_e = pm[:, :D].reshape(bb, F // 2, D)
    em_o = pm[:, D:].reshape(bb, F // 2, D)
    comb = jnp.concatenate([db[:, None, :], em_e, em_o], axis=1)
    g = lax.dot_general(
        comb, comb, (((2,), (2,)), ((0,), (0,))),
        preferred_element_type=f32,
    )
    gf = g.reshape(bb, N * N).astype(bf16)

    h = jnp.dot(db, w1d[...], preferred_element_type=f32)
    h = h + jnp.dot(gf, w1e[...], preferred_element_type=f32)
    h = jax.nn.relu(h + bo1[...])
    h = jax.nn.relu(
        jnp.dot(h.astype(bf16), wo2[...], preferred_element_type=f32) + bo2[...])
    h = jax.nn.relu(
        jnp.dot(h.astype(bf16), wo3[...], preferred_element_type=f32) + bo3[...])
    h = jax.nn.relu(
        jnp.dot(h.astype(bf16), wo4[...], preferred_element_type=f32) + bo4[...])
    out_ref[...] = jnp.sum(h * wo5[...], axis=1, keepdims=True) + bo5[...]


def kernel(dense_features, sparse_indices, offsets, W_embed, dense_params, over_params):
    bf16 = jnp.bfloat16
    flat_idx = (sparse_indices + offsets[None, :]).astype(jnp.int32)
    idx3 = flat_idx.reshape(NW, NCH, CH)
    emb = _sc_gather(W_embed, idx3)
    emb2 = emb.reshape(ROWS // 2, 2 * D)

    dfp = jnp.pad(dense_features, ((0, 0), (0, 3))).astype(bf16)
    (wd1, bd1), (wd2, bd2), (wd3, bd3) = dense_params
    wd1p = jnp.pad(wd1, ((0, 3), (0, 0)))
    (wo1, bo1), (wo2, bo2), (wo3, bo3), (wo4, bo4), (wo5, bo5) = over_params

    w1d = wo1[:D].astype(bf16)
    w1e = jnp.zeros((N * N, wo1.shape[1]), jnp.float32)
    w1e = w1e.at[_SCAT].set(wo1[D:]).astype(bf16)

    bb = 512
    nblk = B // bb
    row2 = lambda a: a.reshape(1, -1)

    grid_spec = pl.GridSpec(
        grid=(nblk,),
        in_specs=[
            pl.BlockSpec((bb, 16), lambda i: (i, 0)),
            pl.BlockSpec((bb * F // 2, 2 * D), lambda i: (i, 0)),
        ]
        + [pl.BlockSpec(s, lambda i: (0, 0)) for s in [
            (16, 512), (1, 512), (512, 256), (1, 256), (256, 64), (1, 64),
            (64, 1024), (N * N, 1024), (1, 1024), (1024, 1024), (1, 1024),
            (1024, 512), (1, 512), (512, 256), (1, 256), (1, 256), (1, 1),
        ]],
        out_specs=pl.BlockSpec((bb, 1), lambda i: (i, 0)),
    )
    logits = pl.pallas_call(
        functools.partial(_tc_body, bb=bb),
        grid_spec=grid_spec,
        out_shape=jax.ShapeDtypeStruct((B, 1), jnp.float32),
        compiler_params=pltpu.CompilerParams(
            dimension_semantics=("arbitrary",),
        ),
    )(
        dfp, emb2,
        wd1p.astype(bf16), row2(bd1), wd2.astype(bf16), row2(bd2),
        wd3.astype(bf16), row2(bd3),
        w1d, w1e, row2(bo1), wo2.astype(bf16), row2(bo2),
        wo3.astype(bf16), row2(bo3), wo4.astype(bf16), row2(bo4),
        row2(wo5[:, 0]), bo5.reshape(1, 1),
    )
    return logits


# speedup vs baseline: 1.6079x; 1.4274x over previous
import functools

import jax
import jax.numpy as jnp
import numpy as np
from jax import lax
from jax.experimental import pallas as pl
from jax.experimental.pallas import tpu as pltpu
from jax.experimental.pallas import tpu_sc as plsc

B = 16384
F = 26
D = 64
N = F + 1
VOCAB = 100000

NC = 2
NS = 16
NW = NC * NS
ROWS = B * F
RPW = ROWS // NW
CH = 128
NCH = RPW // CH
NBUF = 4
NOUT = NCH // NBUF

_TI, _TJ = np.tril_indices(N, -1)
_SCAT = _TI * N + _TJ


def _sc_gather(table, idx3):
    mesh = plsc.VectorSubcoreMesh(core_axis_name="c", subcore_axis_name="s")

    @functools.partial(
        pl.kernel,
        out_type=jax.ShapeDtypeStruct((ROWS, D), jnp.float32),
        mesh=mesh,
        scratch_types=[
            pltpu.VMEM((NCH, CH), jnp.int32),
            pltpu.VMEM((NBUF, CH, D), jnp.float32),
        ]
        + [pltpu.SemaphoreType.DMA] * (2 * NBUF),
        compiler_params=pltpu.CompilerParams(use_tc_tiling_on_sc=True),
    )
    def k(table_hbm, idx_hbm, out_hbm, idx_v, rows_v, *sems):
        gs, ws = sems[:NBUF], sems[NBUF:]
        wid = lax.axis_index("s") * NC + lax.axis_index("c")
        base = wid * RPW
        pltpu.sync_copy(idx_hbm.at[wid], idx_v)

        def start_gather(j, s):
            def group(g, carry):
                t0 = g * 16
                vec = idx_v[j, pl.ds(t0, 16)]
                for tt in range(16):
                    pltpu.async_copy(
                        table_hbm.at[pl.ds(vec[tt], 1)],
                        rows_v.at[s, pl.ds(t0 + tt, 1)],
                        gs[s],
                    )
                return carry

            lax.fori_loop(0, CH // 16, group, 0)

        def wait_gather(s):
            pltpu.make_async_copy(
                table_hbm.at[pl.ds(0, CH)], rows_v.at[s], gs[s]
            ).wait()

        def start_wb(j, s):
            pltpu.async_copy(
                rows_v.at[s], out_hbm.at[pl.ds(base + j * CH, CH)], ws[s]
            )

        def wait_wb(s):
            pltpu.make_async_copy(
                rows_v.at[s], out_hbm.at[pl.ds(0, CH)], ws[s]
            ).wait()

        for s in range(NBUF):
            start_gather(s, s)

        def outer(jj, carry):
            j0 = jj * NBUF
            for s in range(NBUF):
                wait_gather(s)
                start_wb(j0 + s, s)
            for s in range(NBUF):
                wait_wb(s)
                start_gather(j0 + NBUF + s, s)
            return carry

        lax.fori_loop(0, NOUT - 1, outer, 0)

        j0 = (NOUT - 1) * NBUF
        for s in range(NBUF):
            wait_gather(s)
            start_wb(j0 + s, s)
        for s in range(NBUF):
            wait_wb(s)

    return k(table, idx3)


def _tc_body(df_ref, emb_ref, wd1, bd1, wd2, bd2, wd3, bd3,
             w1d, w1e, bo1, wo2, bo2, wo3, bo3, wo4, bo4, wo5, bo5,
             out_ref, *, bb):
    f32 = jnp.float32
    bf16 = jnp.bfloat16
    x = df_ref[...]
    d = jax.nn.relu(jnp.dot(x, wd1[...], preferred_element_type=f32) + bd1[...])
    d = jax.nn.relu(
        jnp.dot(d.astype(bf16), wd2[...], preferred_element_type=f32) + bd2[...])
    d = jax.nn.relu(
        jnp.dot(d.astype(bf16), wd3[...], preferred_element_type=f32) + bd3[...])
    db = d.astype(bf16)

    em = emb_ref[...].astype(bf16)
    emb3 = em.reshape(bb, F, D)
    comb = jnp.concatenate([db[:, None, :], emb3], axis=1)
    g = lax.dot_general(
        comb, comb, (((2,), (2,)), ((0,), (0,))),
        preferred_element_type=f32,
    )
    gf = g.reshape(bb, N * N).astype(bf16)

    h = jnp.dot(db, w1d[...], preferred_element_type=f32)
    h = h + jnp.dot(gf, w1e[...], preferred_element_type=f32)
    h = jax.nn.relu(h + bo1[...])
    h = jax.nn.relu(
        jnp.dot(h.astype(bf16), wo2[...], preferred_element_type=f32) + bo2[...])
    h = jax.nn.relu(
        jnp.dot(h.astype(bf16), wo3[...], preferred_element_type=f32) + bo3[...])
    h = jax.nn.relu(
        jnp.dot(h.astype(bf16), wo4[...], preferred_element_type=f32) + bo4[...])
    out_ref[...] = jnp.sum(h * wo5[...], axis=1, keepdims=True) + bo5[...]


def kernel(dense_features, sparse_indices, offsets, W_embed, dense_params, over_params):
    bf16 = jnp.bfloat16
    flat_idx = (sparse_indices + offsets[None, :]).astype(jnp.int32)
    idx3 = flat_idx.reshape(NW, NCH, CH)
    emb2 = _sc_gather(W_embed, idx3)

    dfp = jnp.pad(dense_features, ((0, 0), (0, 3))).astype(bf16)
    (wd1, bd1), (wd2, bd2), (wd3, bd3) = dense_params
    wd1p = jnp.pad(wd1, ((0, 3), (0, 0)))
    (wo1, bo1), (wo2, bo2), (wo3, bo3), (wo4, bo4), (wo5, bo5) = over_params

    w1d = wo1[:D].astype(bf16)
    w1e = jnp.zeros((N * N, wo1.shape[1]), jnp.float32)
    w1e = w1e.at[_SCAT].set(wo1[D:]).astype(bf16)

    bb = 512
    nblk = B // bb
    row2 = lambda a: a.reshape(1, -1)

    grid_spec = pl.GridSpec(
        grid=(nblk,),
        in_specs=[
            pl.BlockSpec((bb, 16), lambda i: (i, 0)),
            pl.BlockSpec((bb * F, D), lambda i: (i, 0)),
        ]
        + [pl.BlockSpec(s, lambda i: (0, 0)) for s in [
            (16, 512), (1, 512), (512, 256), (1, 256), (256, 64), (1, 64),
            (64, 1024), (N * N, 1024), (1, 1024), (1024, 1024), (1, 1024),
            (1024, 512), (1, 512), (512, 256), (1, 256), (1, 256), (1, 1),
        ]],
        out_specs=pl.BlockSpec((bb, 1), lambda i: (i, 0)),
    )
    logits = pl.pallas_call(
        functools.partial(_tc_body, bb=bb),
        grid_spec=grid_spec,
        out_shape=jax.ShapeDtypeStruct((B, 1), jnp.float32),
        compiler_params=pltpu.CompilerParams(
            dimension_semantics=("arbitrary",),
        ),
    )(
        dfp, emb2,
        wd1p.astype(bf16), row2(bd1), wd2.astype(bf16), row2(bd2),
        wd3.astype(bf16), row2(bd3),
        w1d, w1e, row2(bo1), wo2.astype(bf16), row2(bo2),
        wo3.astype(bf16), row2(bo3), wo4.astype(bf16), row2(bo4),
        row2(wo5[:, 0]), bo5.reshape(1, 1),
    )
    return logits


# speedup vs baseline: 1.6226x; 1.0092x over previous
import functools

import jax
import jax.numpy as jnp
import numpy as np
from jax import lax
from jax.experimental import pallas as pl
from jax.experimental.pallas import tpu as pltpu
from jax.experimental.pallas import tpu_sc as plsc

B = 16384
F = 26
D = 64
N = F + 1
VOCAB = 100000

NC = 2
NS = 16
NW = NC * NS
ROWS = B * F
RPW = ROWS // NW
CH = 128
NCH = RPW // CH
NBUF = 4
NOUT = NCH // NBUF

_TI, _TJ = np.tril_indices(N, -1)
_SCAT = _TI * N + _TJ


def _sc_gather(table, idx3):
    mesh = plsc.VectorSubcoreMesh(core_axis_name="c", subcore_axis_name="s")

    @functools.partial(
        pl.kernel,
        out_type=jax.ShapeDtypeStruct((ROWS, D), jnp.float32),
        mesh=mesh,
        scratch_types=[
            pltpu.VMEM((NCH, CH), jnp.int32),
            pltpu.VMEM((NBUF, CH, D), jnp.float32),
        ]
        + [pltpu.SemaphoreType.DMA] * (2 * NBUF),
        compiler_params=pltpu.CompilerParams(use_tc_tiling_on_sc=True),
    )
    def k(table_hbm, idx_hbm, out_hbm, idx_v, rows_v, *sems):
        gs, ws = sems[:NBUF], sems[NBUF:]
        wid = lax.axis_index("s") * NC + lax.axis_index("c")
        base = wid * RPW
        pltpu.sync_copy(idx_hbm.at[wid], idx_v)

        def start_gather(j, s):
            def group(g, carry):
                t0 = g * 16
                vec = idx_v[j, pl.ds(t0, 16)]
                for tt in range(16):
                    pltpu.async_copy(
                        table_hbm.at[pl.ds(vec[tt], 1)],
                        rows_v.at[s, pl.ds(t0 + tt, 1)],
                        gs[s],
                    )
                return carry

            lax.fori_loop(0, CH // 16, group, 0)

        def wait_gather(s):
            pltpu.make_async_copy(
                table_hbm.at[pl.ds(0, CH)], rows_v.at[s], gs[s]
            ).wait()

        def start_wb(j, s):
            pltpu.async_copy(
                rows_v.at[s], out_hbm.at[pl.ds(base + j * CH, CH)], ws[s]
            )

        def wait_wb(s):
            pltpu.make_async_copy(
                rows_v.at[s], out_hbm.at[pl.ds(0, CH)], ws[s]
            ).wait()

        for s in range(NBUF):
            start_gather(s, s)

        def outer(jj, carry):
            j0 = jj * NBUF
            for s in range(NBUF):
                wait_gather(s)
                start_wb(j0 + s, s)
            for s in range(NBUF):
                wait_wb(s)
                start_gather(j0 + NBUF + s, s)
            return carry

        lax.fori_loop(0, NOUT - 1, outer, 0)

        j0 = (NOUT - 1) * NBUF
        for s in range(NBUF):
            wait_gather(s)
            start_wb(j0 + s, s)
        for s in range(NBUF):
            wait_wb(s)

    return k(table, idx3)


def _tc_body(df_ref, emb_ref, wd1, bd1, wd2, bd2, wd3, bd3,
             w1d, w1e, bo1, wo2, bo2, wo3, bo3, wo4, bo4, wo5, bo5,
             out_ref, *, bb):
    f32 = jnp.float32
    bf16 = jnp.bfloat16
    x = df_ref[...]
    d = jax.nn.relu(jnp.dot(x, wd1[...], preferred_element_type=f32) + bd1[...])
    d = jax.nn.relu(
        jnp.dot(d.astype(bf16), wd2[...], preferred_element_type=f32) + bd2[...])
    d = jax.nn.relu(
        jnp.dot(d.astype(bf16), wd3[...], preferred_element_type=f32) + bd3[...])
    db = d.astype(bf16)

    em = emb_ref[...].astype(bf16)
    emb3 = em.reshape(bb, F, D)
    comb = jnp.concatenate([db[:, None, :], emb3], axis=1)
    g = lax.dot_general(
        comb, comb, (((2,), (2,)), ((0,), (0,))),
        preferred_element_type=f32,
    )
    gf = g.reshape(bb, N * N).astype(bf16)

    h = jnp.dot(db, w1d[...], preferred_element_type=f32)
    h = h + jnp.dot(gf, w1e[...], preferred_element_type=f32)
    h = jax.nn.relu(h + bo1[...])
    h = jax.nn.relu(
        jnp.dot(h.astype(bf16), wo2[...], preferred_element_type=f32) + bo2[...])
    h = jax.nn.relu(
        jnp.dot(h.astype(bf16), wo3[...], preferred_element_type=f32) + bo3[...])
    h = jax.nn.relu(
        jnp.dot(h.astype(bf16), wo4[...], preferred_element_type=f32) + bo4[...])
    out_ref[...] = jnp.sum(h * wo5[...], axis=1, keepdims=True) + bo5[...]


def kernel(dense_features, sparse_indices, offsets, W_embed, dense_params, over_params):
    bf16 = jnp.bfloat16
    flat_idx = (sparse_indices + offsets[None, :]).astype(jnp.int32)
    idx3 = flat_idx.reshape(NW, NCH, CH)
    emb2 = _sc_gather(W_embed, idx3)

    dfp = jnp.pad(dense_features, ((0, 0), (0, 3))).astype(bf16)
    (wd1, bd1), (wd2, bd2), (wd3, bd3) = dense_params
    wd1p = jnp.pad(wd1, ((0, 3), (0, 0)))
    (wo1, bo1), (wo2, bo2), (wo3, bo3), (wo4, bo4), (wo5, bo5) = over_params

    w1d = wo1[:D].astype(bf16)
    w1e = jnp.zeros((N * N, wo1.shape[1]), jnp.float32)
    w1e = w1e.at[_SCAT].set(wo1[D:]).astype(bf16)

    bb = 1024
    nblk = B // bb
    row2 = lambda a: a.reshape(1, -1)

    grid_spec = pl.GridSpec(
        grid=(nblk,),
        in_specs=[
            pl.BlockSpec((bb, 16), lambda i: (i, 0)),
            pl.BlockSpec((bb * F, D), lambda i: (i, 0)),
        ]
        + [pl.BlockSpec(s, lambda i: (0, 0)) for s in [
            (16, 512), (1, 512), (512, 256), (1, 256), (256, 64), (1, 64),
            (64, 1024), (N * N, 1024), (1, 1024), (1024, 1024), (1, 1024),
            (1024, 512), (1, 512), (512, 256), (1, 256), (1, 256), (1, 1),
        ]],
        out_specs=pl.BlockSpec((bb, 1), lambda i: (i, 0)),
    )
    logits = pl.pallas_call(
        functools.partial(_tc_body, bb=bb),
        grid_spec=grid_spec,
        out_shape=jax.ShapeDtypeStruct((B, 1), jnp.float32),
        compiler_params=pltpu.CompilerParams(
            dimension_semantics=("arbitrary",),
        ),
    )(
        dfp, emb2,
        wd1p.astype(bf16), row2(bd1), wd2.astype(bf16), row2(bd2),
        wd3.astype(bf16), row2(bd3),
        w1d, w1e, row2(bo1), wo2.astype(bf16), row2(bo2),
        wo3.astype(bf16), row2(bo3), wo4.astype(bf16), row2(bo4),
        row2(wo5[:, 0]), bo5.reshape(1, 1),
    )
    return logits
